# baseline (speedup 1.0000x reference)
import functools

import jax
import jax.numpy as jnp
from jax import lax
from jax.experimental import pallas as pl
from jax.experimental.pallas import tpu as pltpu
from jax.experimental.pallas import tpu_sc as plsc

D = 16
NC = 2
NS = 16
NW = NC * NS


@functools.partial(jax.jit, static_argnames=("E", "C", "K"))
def _shallow(rx, tx, emb_weight, bias, E, C, K):
    EPW = E // NW
    n_chunks = EPW // C
    G = C // 16
    V = emb_weight.shape[0]
    VPS = V // NS

    mesh = plsc.VectorSubcoreMesh(core_axis_name="c", subcore_axis_name="s")

    @functools.partial(
        pl.kernel,
        mesh=mesh,
        out_type=jax.ShapeDtypeStruct((E,), jnp.float32),
        compiler_params=pltpu.CompilerParams(
            needs_layout_passes=False, use_tc_tiling_on_sc=False),
        scratch_types=[
            pltpu.VMEM((C,), jnp.int32),
            pltpu.VMEM((C,), jnp.int32),
            pltpu.VMEM((C,), jnp.int32),
            pltpu.VMEM((C,), jnp.int32),
            pltpu.VMEM((C, D), jnp.float32),
            pltpu.VMEM((C, D), jnp.float32),
            pltpu.VMEM((C, D), jnp.float32),
            pltpu.VMEM((C, D), jnp.float32),
            pltpu.VMEM((2 * C,), jnp.float32),
            pltpu.VMEM((16,), jnp.float32),
            pltpu.VMEM_SHARED((V, D), jnp.float32),
            pltpu.SemaphoreType.DMA,
            pltpu.SemaphoreType.DMA,
            pltpu.SemaphoreType.DMA,
            pltpu.SemaphoreType.DMA,
        ],
    )
    def k(rx_hbm, tx_hbm, emb_hbm, bias_hbm, out_hbm,
          rxi0, txi0, rxi1, txi1, rxa0, txa0, rxa1, txa1,
          outv, bias_v, emb_sp, sa0, sb0, sa1, sb1):
        sid = lax.axis_index("s")
        wid = sid * NC + lax.axis_index("c")
        pltpu.sync_copy(bias_hbm, bias_v)
        pltpu.sync_copy(emb_hbm.at[pl.ds(sid * VPS, VPS)],
                        emb_sp.at[pl.ds(sid * VPS, VPS)])
        plsc.subcore_barrier()
        lane = lax.iota(jnp.int32, 16)
        b = bias_v[...]
        bufs = ((rxi0, txi0, rxa0, txa0, sa0, sb0),
                (rxi1, txi1, rxa1, txa1, sa1, sb1))

        def issue(ci, po):
            rxi, txi, rxa, txa, sa, sb = bufs[po]
            base = wid * EPW + ci * C
            pltpu.sync_copy(rx_hbm.at[pl.ds(base, C)], rxi)
            pltpu.sync_copy(tx_hbm.at[pl.ds(base, C)], txi)
            ca = pltpu.async_copy(emb_sp.at[rxi], rxa, sa)
            cb = pltpu.async_copy(emb_sp.at[txi], txa, sb)
            return ca, cb

        def compute(ci, po):
            _, _, rxa, txa, _, _ = bufs[po]
            base = wid * EPW + ci * C
            off = (ci % 2) * C

            def group_body(g, c2):
                eidx = g * 16 + lane
                acc = jnp.zeros((16,), jnp.float32)
                for d in range(D):
                    dvec = jnp.full((16,), d, jnp.int32)
                    a = plsc.load_gather(rxa, [eidx, dvec])
                    t = plsc.load_gather(txa, [eidx, dvec])
                    acc = acc + a * t
                logit = acc + b
                outv[pl.ds(off + g * 16, 16)] = 1.0 / (1.0 + jnp.exp(-logit))
                return c2

            lax.fori_loop(0, G, group_body, 0)
            pltpu.sync_copy(outv.at[pl.ds(off, C)], out_hbm.at[pl.ds(base, C)])

        def body(i, carry):
            c0 = K * i
            h = issue(c0, 0)
            for j in range(K):
                h[0].wait()
                h[1].wait()
                if j + 1 < K:
                    h_next = issue(c0 + j + 1, (j + 1) % 2)
                compute(c0 + j, j % 2)
                if j + 1 < K:
                    h = h_next
            return carry

        lax.fori_loop(0, n_chunks // K, body, 0)

    return k(rx, tx, emb_weight, bias)


def kernel(rx, tx, emb_weight, bias):
    bias16 = jnp.broadcast_to(bias.astype(jnp.float32), (16,))
    return _shallow(rx, tx, emb_weight, bias16, rx.shape[0], 400, 5)


# speedup vs baseline: 1.1424x; 1.1424x over previous
import functools

import jax
import jax.numpy as jnp
from jax import lax
from jax.experimental import pallas as pl
from jax.experimental.pallas import tpu as pltpu
from jax.experimental.pallas import tpu_sc as plsc

D = 16
NC = 2
NS = 16
NW = NC * NS


@functools.partial(jax.jit, static_argnames=("E", "C", "K"))
def _shallow(rx, tx, emb_weight, bias, E, C, K):
    EPW = E // NW
    n_chunks = EPW // C
    G = C // 16

    mesh = plsc.VectorSubcoreMesh(core_axis_name="c", subcore_axis_name="s")

    @functools.partial(
        pl.kernel,
        mesh=mesh,
        out_type=jax.ShapeDtypeStruct((E,), jnp.float32),
        compiler_params=pltpu.CompilerParams(
            needs_layout_passes=False, use_tc_tiling_on_sc=False),
        scratch_types=[
            pltpu.VMEM((C,), jnp.int32),
            pltpu.VMEM((C,), jnp.int32),
            pltpu.VMEM((C,), jnp.int32),
            pltpu.VMEM((C,), jnp.int32),
            pltpu.VMEM((C, D), jnp.float32),
            pltpu.VMEM((C, D), jnp.float32),
            pltpu.VMEM((C, D), jnp.float32),
            pltpu.VMEM((C, D), jnp.float32),
            pltpu.VMEM((2 * C,), jnp.float32),
            pltpu.VMEM((16,), jnp.float32),
            pltpu.SemaphoreType.DMA,
            pltpu.SemaphoreType.DMA,
            pltpu.SemaphoreType.DMA,
            pltpu.SemaphoreType.DMA,
        ],
    )
    def k(rx_hbm, tx_hbm, emb_hbm, bias_hbm, out_hbm,
          rxi0, txi0, rxi1, txi1, rxa0, txa0, rxa1, txa1,
          outv, bias_v, sa0, sb0, sa1, sb1):
        wid = lax.axis_index("s") * NC + lax.axis_index("c")
        pltpu.sync_copy(bias_hbm, bias_v)
        lane = lax.iota(jnp.int32, 16)
        b = bias_v[...]
        bufs = ((rxi0, txi0, rxa0, txa0, sa0, sb0),
                (rxi1, txi1, rxa1, txa1, sa1, sb1))

        def issue(ci, po):
            rxi, txi, rxa, txa, sa, sb = bufs[po]
            base = wid * EPW + ci * C
            pltpu.sync_copy(rx_hbm.at[pl.ds(base, C)], rxi)
            pltpu.sync_copy(tx_hbm.at[pl.ds(base, C)], txi)
            ca = pltpu.async_copy(emb_hbm.at[rxi], rxa, sa)
            cb = pltpu.async_copy(emb_hbm.at[txi], txa, sb)
            return ca, cb

        def compute(ci, po):
            _, _, rxa, txa, _, _ = bufs[po]
            base = wid * EPW + ci * C
            off = (ci % 2) * C

            def group_body(g, c2):
                eidx = g * 16 + lane
                acc = jnp.zeros((16,), jnp.float32)
                for d in range(D):
                    dvec = jnp.full((16,), d, jnp.int32)
                    a = plsc.load_gather(rxa, [eidx, dvec])
                    t = plsc.load_gather(txa, [eidx, dvec])
                    acc = acc + a * t
                logit = acc + b
                outv[pl.ds(off + g * 16, 16)] = 1.0 / (1.0 + jnp.exp(-logit))
                return c2

            lax.fori_loop(0, G, group_body, 0)
            pltpu.sync_copy(outv.at[pl.ds(off, C)], out_hbm.at[pl.ds(base, C)])

        def body(i, carry):
            c0 = K * i
            h = issue(c0, 0)
            for j in range(K):
                h[0].wait()
                h[1].wait()
                if j + 1 < K:
                    h_next = issue(c0 + j + 1, (j + 1) % 2)
                compute(c0 + j, j % 2)
                if j + 1 < K:
                    h = h_next
            return carry

        lax.fori_loop(0, n_chunks // K, body, 0)

    return k(rx, tx, emb_weight, bias)


def kernel(rx, tx, emb_weight, bias):
    bias16 = jnp.broadcast_to(bias.astype(jnp.float32), (16,))
    return _shallow(rx, tx, emb_weight, bias16, rx.shape[0], 800, 25)
